# Initial kernel scaffold; baseline (speedup 1.0000x reference)
import jax
import jax.numpy as jnp
from jax.experimental import pallas as pl


def kernel(x, fighter_table, ability_table, W1, b1, W2, b2, Wwin, bwin, Wadv, badv, Wo1, bo1, Wo2, bo2):
    raise NotImplementedError("write your pallas kernel here")


# speedup vs baseline: 4.7918x; 4.7918x over previous
import functools

import jax
import jax.numpy as jnp
from jax.experimental import pallas as pl


def _mish(v):
    sp = jnp.maximum(v, 0.0) + jnp.log1p(jnp.exp(-jnp.abs(v)))
    return v * jnp.tanh(sp)


def _fused_kernel(x_ref, ft_ref, at_ref,
                  w1s_refs, w1d_ref, b1_ref,
                  w2_ref, b2_ref, ww_ref, bw_ref, wa_ref, ba_ref,
                  wo1_ref, bo1_ref, wo2_ref, bo2_ref,
                  ow_ref, oa_ref):
    f32 = jnp.float32
    xb = x_ref[:]
    dense = xb[:, 8:24]

    nt = (((1,), (1,)), ((), ()))
    h1 = jax.lax.dot_general(dense, w1d_ref[:], nt,
                             preferred_element_type=f32)

    iota_f = jax.lax.broadcasted_iota(jnp.int32, (1, 38), 1)
    iota_a = jax.lax.broadcasted_iota(jnp.int32, (1, 241), 1)
    ft = ft_ref[:]
    at = at_ref[:]
    xi = xb[:, 0:8].astype(jnp.int32)
    for s in range(8):
        col = xi[:, s:s + 1]
        if s < 2:
            oh = (col == iota_f).astype(f32)
            emb = jnp.dot(oh, ft, preferred_element_type=f32)
        else:
            oh = (col == iota_a).astype(f32)
            emb = jnp.dot(oh, at, preferred_element_type=f32)
        h1 = h1 + jax.lax.dot_general(emb, w1s_refs[s][:], nt,
                                      preferred_element_type=f32)

    h1 = jnp.tanh(h1 + b1_ref[:])
    h2 = _mish(jax.lax.dot_general(h1, w2_ref[:], nt,
                                   preferred_element_type=f32) + b2_ref[:])
    hw = _mish(jax.lax.dot_general(h2, ww_ref[:], nt,
                                   preferred_element_type=f32) + bw_ref[:])
    ha = _mish(jax.lax.dot_general(h2, wa_ref[:], nt,
                                   preferred_element_type=f32) + ba_ref[:])
    ow = jnp.sum(hw * wo1_ref[:], axis=1, keepdims=True) + bo1_ref[:]
    oa = jnp.sum(ha * wo2_ref[:], axis=1, keepdims=True) + bo2_ref[:]
    ow_ref[:] = jax.nn.sigmoid(ow)
    oa_ref[:] = jnp.tanh(oa)


@functools.partial(jax.jit, static_argnames=("block_b",))
def _run(x, fighter_table, ability_table, w1_parts, b1, W2, b2,
         Wwin, bwin, Wadv, badv, Wo1, bo1, Wo2, bo2, block_b=1024):
    B = x.shape[0]
    rep = lambda *shape: pl.BlockSpec(shape, lambda i: (0,) * len(shape))
    in_specs = (
        [pl.BlockSpec((block_b, 24), lambda i: (i, 0)),
         rep(38, 100), rep(241, 100)]
        + [rep(512, 100)] * 8
        + [rep(512, 16), rep(1, 512), rep(256, 512), rep(1, 256),
           rep(128, 256), rep(1, 128), rep(128, 256), rep(1, 128),
           rep(1, 128), rep(1, 1), rep(1, 128), rep(1, 1)]
    )

    def body(x_ref, ft_ref, at_ref, s0, s1, s2, s3, s4, s5, s6, s7,
             w1d_ref, b1_ref, w2_ref, b2_ref, ww_ref, bw_ref, wa_ref,
             ba_ref, wo1_ref, bo1_ref, wo2_ref, bo2_ref, ow_ref, oa_ref):
        _fused_kernel(x_ref, ft_ref, at_ref,
                      (s0, s1, s2, s3, s4, s5, s6, s7), w1d_ref, b1_ref,
                      w2_ref, b2_ref, ww_ref, bw_ref, wa_ref, ba_ref,
                      wo1_ref, bo1_ref, wo2_ref, bo2_ref, ow_ref, oa_ref)

    out = pl.pallas_call(
        body,
        grid=(B // block_b,),
        in_specs=in_specs,
        out_specs=[pl.BlockSpec((block_b, 1), lambda i: (i, 0))] * 2,
        out_shape=[jax.ShapeDtypeStruct((B, 1), jnp.float32)] * 2,
    )(x, fighter_table, ability_table, *w1_parts, b1, W2, b2,
      Wwin, bwin, Wadv, badv, Wo1, bo1, Wo2, bo2)
    return tuple(out)


def kernel(x, fighter_table, ability_table, W1, b1, W2, b2,
           Wwin, bwin, Wadv, badv, Wo1, bo1, Wo2, bo2):
    w1_parts = tuple(W1[:, s * 100:(s + 1) * 100] for s in range(8))
    w1_parts = w1_parts + (W1[:, 800:816],)
    row = lambda v: v.reshape(1, -1)
    return _run(x, fighter_table, ability_table, w1_parts,
                row(b1), W2, row(b2), Wwin, row(bwin), Wadv, row(badv),
                Wo1, row(bo1), Wo2, row(bo2))
